# baseline (speedup 1.0000x reference)
import functools

import jax
import jax.numpy as jnp
from jax import lax
from jax.experimental import pallas as pl
from jax.experimental.pallas import tpu as pltpu
from jax.experimental.pallas import tpu_sc as plsc

NUM_FIELDS = 26
DIM = 32
B = 16384
NC, NS = 2, 16
NW = NC * NS
CHUNK = 128
ROWS_PER_W = B // NW
CPW = ROWS_PER_W // CHUNK

NBUF = 6
LAG = 3

GROUPS = (26,)


def _group_gather(idxs, tables):
    nf = len(tables)
    ntask = nf * CPW
    mesh = plsc.VectorSubcoreMesh(core_axis_name="c", subcore_axis_name="s")

    @functools.partial(
        pl.kernel,
        out_type=jax.ShapeDtypeStruct((B, nf * DIM), jnp.float32),
        mesh=mesh,
        scratch_types=[
            pltpu.VMEM((nf, CPW, CHUNK), jnp.int32),
            pltpu.VMEM((NBUF, CHUNK, DIM), jnp.float32),
            pltpu.SemaphoreType.DMA((NBUF,)),
            pltpu.SemaphoreType.DMA((NBUF,)),
            pltpu.SemaphoreType.DMA,
        ],
        compiler_params=pltpu.CompilerParams(use_tc_tiling_on_sc=False),
    )
    def k(*refs):
        idx_refs = refs[:nf]
        tab_refs = refs[nf:2 * nf]
        out = refs[2 * nf]
        idx_v, bufs, gsem, ssem, isem = refs[2 * nf + 1:]
        wid = lax.axis_index("s") * NC + lax.axis_index("c")
        b0 = wid * ROWS_PER_W

        icopies = []
        for f in range(nf):
            for c in range(CPW):
                icopies.append(pltpu.async_copy(
                    idx_refs[f].at[pl.ds(b0 + c * CHUNK, CHUNK)],
                    idx_v.at[f, c], isem))
        for h in icopies:
            h.wait()

        ghandles = [None] * NBUF
        shandles = [None] * NBUF

        def start_gather(t):
            f, c = divmod(t, CPW)
            s = t % NBUF
            ghandles[s] = pltpu.async_copy(
                tab_refs[f].at[idx_v.at[f, c]], bufs.at[s], gsem.at[s])

        def retire_gather_start_store(t):
            f, c = divmod(t, CPW)
            s = t % NBUF
            ghandles[s].wait()
            shandles[s] = pltpu.async_copy(
                bufs.at[s],
                out.at[pl.ds(b0 + c * CHUNK, CHUNK), pl.ds(f * DIM, DIM)],
                ssem.at[s])

        for t in range(ntask):
            s = t % NBUF
            if t >= NBUF:
                shandles[s].wait()
            start_gather(t)
            if t >= LAG:
                retire_gather_start_store(t - LAG)
        for u in range(ntask - LAG, ntask):
            retire_gather_start_store(u)
        for u in range(ntask - NBUF, ntask):
            shandles[u % NBUF].wait()

    return k(*idxs, *tables)


from jax.experimental.layout import Format, Layout


@functools.partial(jax.jit, out_shardings=Format(Layout.AUTO))
def _sc_embed(idxs, tables):
    outs = []
    f0 = 0
    for g in GROUPS:
        outs.append(_group_gather(idxs[f0:f0 + g], tables[f0:f0 + g]))
        f0 += g
    out2d = jnp.concatenate(outs, axis=1) if len(outs) > 1 else outs[0]
    return out2d.reshape(B, NUM_FIELDS, DIM)


def kernel(f00, f01, f02, f03, f04, f05, f06, f07, f08, f09, f10, f11, f12,
           f13, f14, f15, f16, f17, f18, f19, f20, f21, f22, f23, f24, f25,
           W00, W01, W02, W03, W04, W05, W06, W07, W08, W09, W10, W11, W12,
           W13, W14, W15, W16, W17, W18, W19, W20, W21, W22, W23, W24, W25):
    idxs = [f00, f01, f02, f03, f04, f05, f06, f07, f08, f09, f10, f11, f12,
            f13, f14, f15, f16, f17, f18, f19, f20, f21, f22, f23, f24, f25]
    tables = [W00, W01, W02, W03, W04, W05, W06, W07, W08, W09, W10, W11, W12,
              W13, W14, W15, W16, W17, W18, W19, W20, W21, W22, W23, W24, W25]
    return _sc_embed(idxs, tables)


# speedup vs baseline: 1.0003x; 1.0003x over previous
import functools

import jax
import jax.numpy as jnp
from jax import lax
from jax.experimental import pallas as pl
from jax.experimental.pallas import tpu as pltpu
from jax.experimental.pallas import tpu_sc as plsc

NUM_FIELDS = 26
DIM = 32
B = 16384
NC, NS = 2, 16
NW = NC * NS
CHUNK = 128
ROWS_PER_W = B // NW
CPW = ROWS_PER_W // CHUNK

NBUF = 6
LAG = 3

GROUPS = (26,)


def _group_gather(idxs, tables):
    nf = len(tables)
    ntask = nf * CPW
    mesh = plsc.VectorSubcoreMesh(core_axis_name="c", subcore_axis_name="s")

    @functools.partial(
        pl.kernel,
        out_type=jax.ShapeDtypeStruct((B, nf * DIM), jnp.float32),
        mesh=mesh,
        scratch_types=[
            pltpu.VMEM((nf, CPW, CHUNK), jnp.int32),
            pltpu.VMEM((NBUF, CHUNK, DIM), jnp.float32),
            pltpu.SemaphoreType.DMA((NBUF,)),
            pltpu.SemaphoreType.DMA((NBUF,)),
            pltpu.SemaphoreType.DMA,
        ],
        compiler_params=pltpu.CompilerParams(use_tc_tiling_on_sc=False),
    )
    def k(*refs):
        idx_refs = refs[:nf]
        tab_refs = refs[nf:2 * nf]
        out = refs[2 * nf]
        idx_v, bufs, gsem, ssem, isem = refs[2 * nf + 1:]
        wid = lax.axis_index("s") * NC + lax.axis_index("c")
        b0 = wid * ROWS_PER_W

        icopies = []
        for f in range(nf):
            for c in range(CPW):
                icopies.append(pltpu.async_copy(
                    idx_refs[f].at[pl.ds(b0 + c * CHUNK, CHUNK)],
                    idx_v.at[f, c], isem))
        for h in icopies:
            h.wait()

        ghandles = [None] * NBUF
        shandles = [None] * NBUF

        def start_gather(t):
            f, c = divmod(t, CPW)
            s = t % NBUF
            ghandles[s] = pltpu.async_copy(
                tab_refs[f].at[idx_v.at[f, c]], bufs.at[s], gsem.at[s])

        def retire_gather_start_store(t):
            f, c = divmod(t, CPW)
            s = t % NBUF
            ghandles[s].wait()
            shandles[s] = pltpu.async_copy(
                bufs.at[s],
                out.at[pl.ds(b0 + c * CHUNK, CHUNK), pl.ds(f * DIM, DIM)],
                ssem.at[s])

        for t in range(ntask):
            s = t % NBUF
            if t >= NBUF:
                shandles[s].wait()
            start_gather(t)
            if t >= LAG:
                retire_gather_start_store(t - LAG)
        for u in range(ntask - LAG, ntask):
            retire_gather_start_store(u)
        for u in range(ntask - NBUF, ntask):
            shandles[u % NBUF].wait()

    return k(*idxs, *tables)


@jax.jit
def _sc_embed(idxs, tables):
    outs = []
    f0 = 0
    for g in GROUPS:
        outs.append(_group_gather(idxs[f0:f0 + g], tables[f0:f0 + g]))
        f0 += g
    out2d = jnp.concatenate(outs, axis=1) if len(outs) > 1 else outs[0]
    return out2d.reshape(B, NUM_FIELDS, DIM)


def kernel(f00, f01, f02, f03, f04, f05, f06, f07, f08, f09, f10, f11, f12,
           f13, f14, f15, f16, f17, f18, f19, f20, f21, f22, f23, f24, f25,
           W00, W01, W02, W03, W04, W05, W06, W07, W08, W09, W10, W11, W12,
           W13, W14, W15, W16, W17, W18, W19, W20, W21, W22, W23, W24, W25):
    idxs = [f00, f01, f02, f03, f04, f05, f06, f07, f08, f09, f10, f11, f12,
            f13, f14, f15, f16, f17, f18, f19, f20, f21, f22, f23, f24, f25]
    tables = [W00, W01, W02, W03, W04, W05, W06, W07, W08, W09, W10, W11, W12,
              W13, W14, W15, W16, W17, W18, W19, W20, W21, W22, W23, W24, W25]
    return _sc_embed(idxs, tables)


# speedup vs baseline: 1.0021x; 1.0018x over previous
import functools

import jax
import jax.numpy as jnp
from jax import lax
from jax.experimental import pallas as pl
from jax.experimental.pallas import tpu as pltpu
from jax.experimental.pallas import tpu_sc as plsc

NUM_FIELDS = 26
DIM = 32
B = 16384
NC, NS = 2, 16
NW = NC * NS
CHUNK = 128
ROWS_PER_W = B // NW
CPW = ROWS_PER_W // CHUNK

NBUF = 12
LAG = 6

GROUPS = (26,)


def _group_gather(idxs, tables):
    nf = len(tables)
    ntask = nf * CPW
    mesh = plsc.VectorSubcoreMesh(core_axis_name="c", subcore_axis_name="s")

    @functools.partial(
        pl.kernel,
        out_type=jax.ShapeDtypeStruct((B, nf * DIM), jnp.float32),
        mesh=mesh,
        scratch_types=[
            pltpu.VMEM((nf, CPW, CHUNK), jnp.int32),
            pltpu.VMEM((NBUF, CHUNK, DIM), jnp.float32),
            pltpu.SemaphoreType.DMA((NBUF,)),
            pltpu.SemaphoreType.DMA((NBUF,)),
            pltpu.SemaphoreType.DMA,
        ],
        compiler_params=pltpu.CompilerParams(use_tc_tiling_on_sc=False),
    )
    def k(*refs):
        idx_refs = refs[:nf]
        tab_refs = refs[nf:2 * nf]
        out = refs[2 * nf]
        idx_v, bufs, gsem, ssem, isem = refs[2 * nf + 1:]
        wid = lax.axis_index("s") * NC + lax.axis_index("c")
        b0 = wid * ROWS_PER_W

        icopies = []
        for f in range(nf):
            for c in range(CPW):
                icopies.append(pltpu.async_copy(
                    idx_refs[f].at[pl.ds(b0 + c * CHUNK, CHUNK)],
                    idx_v.at[f, c], isem))
        for h in icopies:
            h.wait()

        ghandles = [None] * NBUF
        shandles = [None] * NBUF

        def start_gather(t):
            f, c = divmod(t, CPW)
            s = t % NBUF
            ghandles[s] = pltpu.async_copy(
                tab_refs[f].at[idx_v.at[f, c]], bufs.at[s], gsem.at[s])

        def retire_gather_start_store(t):
            f, c = divmod(t, CPW)
            s = t % NBUF
            ghandles[s].wait()
            shandles[s] = pltpu.async_copy(
                bufs.at[s],
                out.at[pl.ds(b0 + c * CHUNK, CHUNK), pl.ds(f * DIM, DIM)],
                ssem.at[s])

        for t in range(ntask):
            s = t % NBUF
            if t >= NBUF:
                shandles[s].wait()
            start_gather(t)
            if t >= LAG:
                retire_gather_start_store(t - LAG)
        for u in range(ntask - LAG, ntask):
            retire_gather_start_store(u)
        for u in range(ntask - NBUF, ntask):
            shandles[u % NBUF].wait()

    return k(*idxs, *tables)


@jax.jit
def _sc_embed(idxs, tables):
    outs = []
    f0 = 0
    for g in GROUPS:
        outs.append(_group_gather(idxs[f0:f0 + g], tables[f0:f0 + g]))
        f0 += g
    out2d = jnp.concatenate(outs, axis=1) if len(outs) > 1 else outs[0]
    return out2d.reshape(B, NUM_FIELDS, DIM)


def kernel(f00, f01, f02, f03, f04, f05, f06, f07, f08, f09, f10, f11, f12,
           f13, f14, f15, f16, f17, f18, f19, f20, f21, f22, f23, f24, f25,
           W00, W01, W02, W03, W04, W05, W06, W07, W08, W09, W10, W11, W12,
           W13, W14, W15, W16, W17, W18, W19, W20, W21, W22, W23, W24, W25):
    idxs = [f00, f01, f02, f03, f04, f05, f06, f07, f08, f09, f10, f11, f12,
            f13, f14, f15, f16, f17, f18, f19, f20, f21, f22, f23, f24, f25]
    tables = [W00, W01, W02, W03, W04, W05, W06, W07, W08, W09, W10, W11, W12,
              W13, W14, W15, W16, W17, W18, W19, W20, W21, W22, W23, W24, W25]
    return _sc_embed(idxs, tables)
